# baseline (speedup 1.0000x reference)
import jax
import jax.numpy as jnp
from jax import lax
from jax.experimental import pallas as pl
from jax.experimental.pallas import tpu as pltpu
from jax.experimental.pallas import tpu_sc as plsc

NC = 2
NS = 16
L = 16
NW = NC * NS
ROWS = 128
COLS = 32768
SC_RPW = 2
SC_ROWS = NW * SC_RPW
TC_ROWS = ROWS - SC_ROWS
NBLK = COLS // L
U = 8
NIT = NBLK // U
BIG = 2**31 - 1
TC_BR = 8


def _row_argmax(buf):
    iota = lax.iota(jnp.int32, L)
    init_max = tuple(jnp.full((L,), -jnp.inf, jnp.float32) for _ in range(U))
    init_blk = tuple(jnp.zeros((L,), jnp.int32) for _ in range(U))

    def step(i, carry):
        maxs, blks = carry
        base = i * (U * L)
        new_maxs = []
        new_blks = []
        for u in range(U):
            chunk = buf[pl.ds(base + u * L, L)]
            m = chunk > maxs[u]
            new_maxs.append(jnp.where(m, chunk, maxs[u]))
            new_blks.append(jnp.where(m, i, blks[u]))
        return tuple(new_maxs), tuple(new_blks)

    maxs, blks = lax.fori_loop(0, NIT, step, (init_max, init_blk))

    vmax = maxs[0]
    vpos = blks[0] * (U * L) + iota
    for u in range(1, U):
        pu = blks[u] * (U * L) + (u * L) + iota
        better = (maxs[u] > vmax) | ((maxs[u] == vmax) & (pu < vpos))
        vmax = jnp.where(better, maxs[u], vmax)
        vpos = jnp.where(better, pu, vpos)

    gmax = jnp.max(vmax)
    cand = jnp.where(vmax == gmax, vpos, BIG)
    return jnp.min(cand)


def _sc_body(in_hbm, out_hbm, buf0, buf1, res_buf, sem0, sem1):
    wid = lax.axis_index("s") * NC + lax.axis_index("c")
    row0 = TC_ROWS + wid * SC_RPW

    bufs = (buf0, buf1)
    sems = (sem0, sem1)
    pltpu.make_async_copy(in_hbm.at[row0], buf0, sem0).start()

    res = jnp.zeros((L,), jnp.int32)
    iota = lax.iota(jnp.int32, L)
    for r in range(SC_RPW):
        buf = bufs[r % 2]
        sem = sems[r % 2]
        if r + 1 < SC_RPW:
            pltpu.make_async_copy(
                in_hbm.at[row0 + r + 1], bufs[(r + 1) % 2], sems[(r + 1) % 2]
            ).start()
        pltpu.make_async_copy(in_hbm.at[row0 + r], buf, sem).wait()
        idx = _row_argmax(buf)
        res = jnp.where(iota == r, idx, res)

    res_buf[...] = res
    pltpu.sync_copy(res_buf, out_hbm.at[wid])


def _tc_body(x_ref, o_ref):
    x = x_ref[...]
    m = jnp.max(x, axis=1, keepdims=True)
    iota = lax.broadcasted_iota(jnp.int32, x.shape, 1)
    idx = jnp.min(jnp.where(x == m, iota, BIG), axis=1, keepdims=True)
    o_ref[...] = jnp.broadcast_to(idx, (TC_BR, 128))


@jax.jit
def kernel(input):
    mesh = plsc.VectorSubcoreMesh(core_axis_name="c", subcore_axis_name="s")
    sc = pl.kernel(
        _sc_body,
        out_type=jax.ShapeDtypeStruct((NW, L), jnp.int32),
        mesh=mesh,
        scratch_types=[
            pltpu.VMEM((COLS,), jnp.float32),
            pltpu.VMEM((COLS,), jnp.float32),
            pltpu.VMEM((L,), jnp.int32),
            pltpu.SemaphoreType.DMA,
            pltpu.SemaphoreType.DMA,
        ],
        compiler_params=pltpu.CompilerParams(needs_layout_passes=False),
    )
    sc_packed = sc(input)

    tc = pl.pallas_call(
        _tc_body,
        grid=(TC_ROWS // TC_BR,),
        in_specs=[pl.BlockSpec((TC_BR, COLS), lambda i: (i, 0))],
        out_specs=pl.BlockSpec((TC_BR, 128), lambda i: (i, 0)),
        out_shape=jax.ShapeDtypeStruct((TC_ROWS, 128), jnp.int32),
    )
    tc_packed = tc(input[:TC_ROWS])

    tc_out = tc_packed[:, 0]
    sc_out = sc_packed[:, :SC_RPW].reshape(SC_ROWS)
    return jnp.concatenate([tc_out, sc_out])


# speedup vs baseline: 1.3009x; 1.3009x over previous
import jax
import jax.numpy as jnp
from jax import lax
from jax.experimental import pallas as pl
from jax.experimental.pallas import tpu as pltpu
from jax.experimental.pallas import tpu_sc as plsc

NC = 2
NS = 16
L = 16
NW = NC * NS
ROWS = 128
COLS = 32768
SC_RPW = 2
SC_ROWS = NW * SC_RPW
TC_ROWS = ROWS - SC_ROWS
NBLK = COLS // L
U = 8
NIT = NBLK // U
BIG = 2**31 - 1
TC_BR = 8


def _row_argmax(buf):
    iota = lax.iota(jnp.int32, L)
    init_max = tuple(jnp.full((L,), -jnp.inf, jnp.float32) for _ in range(U))
    init_blk = tuple(jnp.zeros((L,), jnp.int32) for _ in range(U))

    def step(i, carry):
        maxs, blks = carry
        base = i * (U * L)
        new_maxs = []
        new_blks = []
        for u in range(U):
            chunk = buf[pl.ds(base + u * L, L)]
            m = chunk > maxs[u]
            new_maxs.append(jnp.where(m, chunk, maxs[u]))
            new_blks.append(jnp.where(m, i, blks[u]))
        return tuple(new_maxs), tuple(new_blks)

    maxs, blks = lax.fori_loop(0, NIT, step, (init_max, init_blk))

    vmax = maxs[0]
    vpos = blks[0] * (U * L) + iota
    for u in range(1, U):
        pu = blks[u] * (U * L) + (u * L) + iota
        better = (maxs[u] > vmax) | ((maxs[u] == vmax) & (pu < vpos))
        vmax = jnp.where(better, maxs[u], vmax)
        vpos = jnp.where(better, pu, vpos)

    gmax = jnp.max(vmax)
    cand = jnp.where(vmax == gmax, vpos, BIG)
    return jnp.min(cand)


def _sc_body(in_hbm, out_hbm, buf0, buf1, res_buf, sem0, sem1):
    wid = lax.axis_index("s") * NC + lax.axis_index("c")
    row0 = TC_ROWS + wid * SC_RPW

    bufs = (buf0, buf1)
    sems = (sem0, sem1)
    pltpu.make_async_copy(in_hbm.at[row0], buf0, sem0).start()

    res = jnp.zeros((L,), jnp.int32)
    iota = lax.iota(jnp.int32, L)
    for r in range(SC_RPW):
        buf = bufs[r % 2]
        sem = sems[r % 2]
        if r + 1 < SC_RPW:
            pltpu.make_async_copy(
                in_hbm.at[row0 + r + 1], bufs[(r + 1) % 2], sems[(r + 1) % 2]
            ).start()
        pltpu.make_async_copy(in_hbm.at[row0 + r], buf, sem).wait()
        idx = _row_argmax(buf)
        res = jnp.where(iota == r, idx, res)

    res_buf[...] = res
    pltpu.sync_copy(res_buf, out_hbm.at[wid])


def _tc_body(x_ref, o_ref):
    x = x_ref[...]
    m = jnp.max(x, axis=1, keepdims=True)
    iota = lax.broadcasted_iota(jnp.int32, x.shape, 1)
    idx = jnp.min(jnp.where(x == m, iota, BIG), axis=1, keepdims=True)
    o_ref[...] = jnp.broadcast_to(idx, (TC_BR, 128))


@jax.jit
def kernel(input):
    mesh = plsc.VectorSubcoreMesh(core_axis_name="c", subcore_axis_name="s")
    sc = pl.kernel(
        _sc_body,
        out_type=jax.ShapeDtypeStruct((NW, L), jnp.int32),
        mesh=mesh,
        scratch_types=[
            pltpu.VMEM((COLS,), jnp.float32),
            pltpu.VMEM((COLS,), jnp.float32),
            pltpu.VMEM((L,), jnp.int32),
            pltpu.SemaphoreType.DMA,
            pltpu.SemaphoreType.DMA,
        ],
        compiler_params=pltpu.CompilerParams(needs_layout_passes=False),
    )
    sc_packed = sc(input)

    tc = pl.pallas_call(
        _tc_body,
        grid=(TC_ROWS // TC_BR,),
        in_specs=[pl.BlockSpec((TC_BR, COLS), lambda i: (i, 0))],
        out_specs=pl.BlockSpec((TC_BR, 128), lambda i: (i, 0)),
        out_shape=jax.ShapeDtypeStruct((TC_ROWS, 128), jnp.int32),
    )
    tc_packed = tc(input)

    tc_out = tc_packed[:, 0]
    sc_out = sc_packed[:, :SC_RPW].reshape(SC_ROWS)
    return jnp.concatenate([tc_out, sc_out])
